# baseline (speedup 1.0000x reference)
import functools

import jax
import jax.numpy as jnp
from jax import lax
from jax.experimental import pallas as pl
from jax.experimental.pallas import tpu as pltpu
from jax.experimental.pallas import tpu_sc as plsc

_Q = 0.7
_K = 0.5
_B = 16384
_NCLS = 1000
_NCORES = 2
_NSUB = 16
_NW = _NCORES * _NSUB
_PER_W = _B // _NW
_CHUNK = 128
_NCH = _PER_W // _CHUNK
_NBLK = _B // _CHUNK
_GRID = 8
_CONST = (1.0 - _K ** _Q) / _Q


def _sc_gather_blocks(lt, targets):
    mesh = plsc.VectorSubcoreMesh(core_axis_name="c", subcore_axis_name="s")

    @functools.partial(
        pl.kernel,
        mesh=mesh,
        out_type=jax.ShapeDtypeStruct((_NBLK, _CHUNK, _CHUNK), jnp.float32),
        scratch_types=[
            pltpu.VMEM((_PER_W,), jnp.int32),
            pltpu.VMEM((_CHUNK, _CHUNK), jnp.float32),
            pltpu.VMEM((_CHUNK, _CHUNK), jnp.float32),
            pltpu.VMEM((_CHUNK, _CHUNK), jnp.float32),
            pltpu.VMEM((_CHUNK, _CHUNK), jnp.float32),
            pltpu.SemaphoreType.DMA,
            pltpu.SemaphoreType.DMA,
        ],
    )
    def blocks_kernel(lt_hbm, t_hbm, blk_out, tgt_v, b0, b1, b2, b3,
                      sem_g, sem_o):
        wid = lax.axis_index("c") * _NSUB + lax.axis_index("s")
        base = wid * _PER_W
        pltpu.sync_copy(t_hbm.at[pl.ds(base, _PER_W)], tgt_v)
        bufs = (b0, b1, b2, b3)
        gcopies = []
        for g in range(_NCH):
            gcopies.append(pltpu.async_copy(
                lt_hbm.at[tgt_v.at[pl.ds(g * _CHUNK, _CHUNK)],
                          pl.ds(base + g * _CHUNK, _CHUNK)],
                bufs[g], sem_g))
        ocopies = []
        for g in range(_NCH):
            gcopies[g].wait()
            ocopies.append(pltpu.async_copy(
                bufs[g], blk_out.at[wid * _NCH + g], sem_o))
        for cp in ocopies:
            cp.wait()

    return blocks_kernel(lt, targets)


def _sc_gather_w(weight_flat, indexes):
    mesh = plsc.VectorSubcoreMesh(core_axis_name="c", subcore_axis_name="s")

    @functools.partial(
        pl.kernel,
        mesh=mesh,
        out_type=jax.ShapeDtypeStruct((_B,), jnp.float32),
        scratch_types=[
            pltpu.VMEM((_PER_W,), jnp.int32),
            pltpu.VMEM((_PER_W,), jnp.float32),
            pltpu.SemaphoreType.DMA,
        ],
    )
    def gather_kernel(w_hbm, i_hbm, w_out, widx_v, w_v, sem):
        wid = lax.axis_index("c") * _NSUB + lax.axis_index("s")
        base = wid * _PER_W
        pltpu.sync_copy(i_hbm.at[pl.ds(base, _PER_W)], widx_v)
        copies = []
        for c in range(_NCH):
            sl = pl.ds(c * _CHUNK, _CHUNK)
            copies.append(pltpu.async_copy(
                w_hbm.at[widx_v.at[sl]], w_v.at[sl], sem))
        for cp in copies:
            cp.wait()
        pltpu.sync_copy(w_v, w_out.at[pl.ds(base, _PER_W)])

    return gather_kernel(weight_flat, indexes)


def _loss_body(eye_ref, blk_ref, w_ref, out_ref):
    i = pl.program_id(0)
    blk = blk_ref[...]
    w = w_ref[...]
    yg = jnp.sum(blk * eye_ref[...][None, :, :], axis=2)
    p = jnp.exp(jnp.log(yg) * _Q)
    part = jnp.sum(((1.0 - p) * (1.0 / _Q) - _CONST) * w) * (1.0 / _B)

    @pl.when(i == 0)
    def _():
        out_ref[0, 0] = 0.0

    out_ref[0, 0] += part


def kernel(logits, targets, indexes, weight):
    idx = indexes.astype(jnp.int32)
    tgt = targets.astype(jnp.int32)
    lt = logits.T
    blk = _sc_gather_blocks(lt, tgt)
    idx_dep, _ = jax.lax.optimization_barrier((idx, blk))
    w = _sc_gather_w(weight.reshape(-1), idx_dep)
    out = pl.pallas_call(
        _loss_body,
        grid=(_GRID,),
        in_specs=[
            pl.BlockSpec((_CHUNK, _CHUNK), lambda i: (0, 0)),
            pl.BlockSpec((_NBLK // _GRID, _CHUNK, _CHUNK), lambda i: (i, 0, 0)),
            pl.BlockSpec((_NBLK // _GRID, _CHUNK), lambda i: (i, 0)),
        ],
        out_specs=pl.BlockSpec((1, 1), lambda i: (0, 0),
                               memory_space=pltpu.SMEM),
        out_shape=jax.ShapeDtypeStruct((1, 1), jnp.float32),
    )(jnp.eye(_CHUNK, dtype=jnp.float32), blk, w.reshape(_NBLK, _CHUNK))
    return out[0, 0]


# speedup vs baseline: 1.4444x; 1.4444x over previous
import functools

import jax
import jax.numpy as jnp
from jax import lax
from jax.experimental import pallas as pl
from jax.experimental.pallas import tpu as pltpu
from jax.experimental.pallas import tpu_sc as plsc

_Q = 0.7
_K = 0.5
_B = 16384
_NCLS = 1000
_NCORES = 2
_NSUB = 16
_NW = _NCORES * _NSUB
_PER_W = _B // _NW
_CHUNK = 128
_NCH = _PER_W // _CHUNK
_NBLK = _B // _CHUNK
_GRID = 8
_CONST = (1.0 - _K ** _Q) / _Q


def _sc_gather_blocks(lt, targets):
    mesh = plsc.VectorSubcoreMesh(core_axis_name="c", subcore_axis_name="s")

    @functools.partial(
        pl.kernel,
        mesh=mesh,
        out_type=jax.ShapeDtypeStruct((_NBLK, _CHUNK, _CHUNK), jnp.float32),
        scratch_types=[
            pltpu.VMEM((_PER_W,), jnp.int32),
            pltpu.VMEM((_CHUNK, _CHUNK), jnp.float32),
            pltpu.VMEM((_CHUNK, _CHUNK), jnp.float32),
            pltpu.VMEM((_CHUNK, _CHUNK), jnp.float32),
            pltpu.VMEM((_CHUNK, _CHUNK), jnp.float32),
            pltpu.SemaphoreType.DMA,
            pltpu.SemaphoreType.DMA,
        ],
    )
    def blocks_kernel(lt_hbm, t_hbm, blk_out, tgt_v, b0, b1, b2, b3,
                      sem_g, sem_o):
        wid = lax.axis_index("c") * _NSUB + lax.axis_index("s")
        base = wid * _PER_W
        pltpu.sync_copy(t_hbm.at[pl.ds(base, _PER_W)], tgt_v)
        bufs = (b0, b1, b2, b3)
        gcopies = []
        for g in range(_NCH):
            gcopies.append(pltpu.async_copy(
                lt_hbm.at[tgt_v.at[pl.ds(g * _CHUNK, _CHUNK)],
                          pl.ds(base + g * _CHUNK, _CHUNK)],
                bufs[g], sem_g))
        ocopies = []
        for g in range(_NCH):
            gcopies[g].wait()
            ocopies.append(pltpu.async_copy(
                bufs[g], blk_out.at[wid * _NCH + g], sem_o))
        for cp in ocopies:
            cp.wait()

    return blocks_kernel(lt, targets)


_TRAIN = 1000000
_STAGE = 62504


def _sc_gather_w(weight_row, indexes):
    mesh = plsc.VectorSubcoreMesh(core_axis_name="c", subcore_axis_name="s")

    @functools.partial(
        pl.kernel,
        mesh=mesh,
        out_type=jax.ShapeDtypeStruct((_B,), jnp.float32),
        scratch_types=[
            pltpu.VMEM((_PER_W,), jnp.int32),
            pltpu.VMEM((_PER_W,), jnp.float32),
            pltpu.VMEM_SHARED((_TRAIN,), jnp.float32),
            pltpu.SemaphoreType.DMA,
        ],
    )
    def gather_kernel(w_hbm, i_hbm, w_out, widx_v, w_v, w_sh, sem):
        wid = lax.axis_index("c") * _NSUB + lax.axis_index("s")
        base = wid * _PER_W
        pltpu.sync_copy(i_hbm.at[pl.ds(base, _PER_W)], widx_v)
        s = lax.axis_index("s")

        @pl.when(s == 0)
        def _():
            pltpu.sync_copy(w_hbm.at[0], w_sh)

        plsc.subcore_barrier()
        copies = []
        for c in range(_NCH):
            sl = pl.ds(c * _CHUNK, _CHUNK)
            copies.append(pltpu.async_copy(
                w_sh.at[widx_v.at[sl]], w_v.at[sl], sem))
        for cp in copies:
            cp.wait()
        pltpu.sync_copy(w_v, w_out.at[pl.ds(base, _PER_W)])

    return gather_kernel(weight_row, indexes)


def _loss_body(eye_ref, blk_ref, w_ref, out_ref):
    i = pl.program_id(0)
    blk = blk_ref[...]
    w = w_ref[...]
    yg = jnp.sum(blk * eye_ref[...][None, :, :], axis=2)
    p = jnp.exp(jnp.log(yg) * _Q)
    part = jnp.sum(((1.0 - p) * (1.0 / _Q) - _CONST) * w) * (1.0 / _B)

    @pl.when(i == 0)
    def _():
        out_ref[0, 0] = 0.0

    out_ref[0, 0] += part


def kernel(logits, targets, indexes, weight):
    idx = indexes.astype(jnp.int32)
    tgt = targets.astype(jnp.int32)
    lt = logits.T
    blk = _sc_gather_blocks(lt, tgt)
    idx_dep, _ = jax.lax.optimization_barrier((idx, blk))
    w = _sc_gather_w(weight.T, idx_dep)
    out = pl.pallas_call(
        _loss_body,
        grid=(_GRID,),
        in_specs=[
            pl.BlockSpec((_CHUNK, _CHUNK), lambda i: (0, 0)),
            pl.BlockSpec((_NBLK // _GRID, _CHUNK, _CHUNK), lambda i: (i, 0, 0)),
            pl.BlockSpec((_NBLK // _GRID, _CHUNK), lambda i: (i, 0)),
        ],
        out_specs=pl.BlockSpec((1, 1), lambda i: (0, 0),
                               memory_space=pltpu.SMEM),
        out_shape=jax.ShapeDtypeStruct((1, 1), jnp.float32),
    )(jnp.eye(_CHUNK, dtype=jnp.float32), blk, w.reshape(_NBLK, _CHUNK))
    return out[0, 0]


# speedup vs baseline: 1.7686x; 1.2245x over previous
import functools

import jax
import jax.numpy as jnp
from jax import lax
from jax.experimental import pallas as pl
from jax.experimental.pallas import tpu as pltpu
from jax.experimental.pallas import tpu_sc as plsc

_Q = 0.7
_K = 0.5
_B = 16384
_NCLS = 1000
_NCORES = 2
_NSUB = 16
_NW = _NCORES * _NSUB
_PER_W = _B // _NW
_CHUNK = 128
_NCH = _PER_W // _CHUNK
_NBLK = _B // _CHUNK
_GRID = 8
_TRAIN = 1000000
_CONST = (1.0 - _K ** _Q) / _Q


def _sc_gather(lt, weight_row, targets, indexes):
    mesh = plsc.VectorSubcoreMesh(core_axis_name="c", subcore_axis_name="s")

    @functools.partial(
        pl.kernel,
        mesh=mesh,
        out_type=(
            jax.ShapeDtypeStruct((_NBLK, _CHUNK, _CHUNK), jnp.float32),
            jax.ShapeDtypeStruct((_B,), jnp.float32),
        ),
        scratch_types=[
            pltpu.VMEM((_PER_W,), jnp.int32),
            pltpu.VMEM((_PER_W,), jnp.int32),
            pltpu.VMEM((_CHUNK, _CHUNK), jnp.float32),
            pltpu.VMEM((_CHUNK, _CHUNK), jnp.float32),
            pltpu.VMEM((_CHUNK, _CHUNK), jnp.float32),
            pltpu.VMEM((_CHUNK, _CHUNK), jnp.float32),
            pltpu.VMEM((_PER_W,), jnp.float32),
            pltpu.VMEM_SHARED((_TRAIN,), jnp.float32),
            pltpu.SemaphoreType.DMA,
            pltpu.SemaphoreType.DMA,
            pltpu.SemaphoreType.DMA,
        ],
    )
    def gather_kernel(lt_hbm, w_hbm, t_hbm, i_hbm, blk_out, w_out,
                      tgt_v, widx_v, b0, b1, b2, b3, w_v, w_sh,
                      sem_g, sem_o, sem_w):
        wid = lax.axis_index("c") * _NSUB + lax.axis_index("s")
        base = wid * _PER_W
        pltpu.sync_copy(t_hbm.at[pl.ds(base, _PER_W)], tgt_v)
        pltpu.sync_copy(i_hbm.at[pl.ds(base, _PER_W)], widx_v)
        bufs = (b0, b1, b2, b3)
        gcopies = []
        for g in range(_NCH):
            gcopies.append(pltpu.async_copy(
                lt_hbm.at[tgt_v.at[pl.ds(g * _CHUNK, _CHUNK)],
                          pl.ds(base + g * _CHUNK, _CHUNK)],
                bufs[g], sem_g))

        @pl.when(lax.axis_index("s") == 0)
        def _():
            pltpu.sync_copy(w_hbm.at[0], w_sh)

        plsc.subcore_barrier()
        wcopies = []
        for c in range(_NCH):
            sl = pl.ds(c * _CHUNK, _CHUNK)
            wcopies.append(pltpu.async_copy(
                w_sh.at[widx_v.at[sl]], w_v.at[sl], sem_w))
        ocopies = []
        for g in range(_NCH):
            gcopies[g].wait()
            ocopies.append(pltpu.async_copy(
                bufs[g], blk_out.at[wid * _NCH + g], sem_o))
        for cp in wcopies:
            cp.wait()
        pltpu.sync_copy(w_v, w_out.at[pl.ds(base, _PER_W)])
        for cp in ocopies:
            cp.wait()

    return gather_kernel(lt, weight_row, targets, indexes)


def _loss_body(eye_ref, blk_ref, w_ref, out_ref):
    i = pl.program_id(0)
    blk = blk_ref[...]
    w = w_ref[...]
    yg = jnp.sum(blk * eye_ref[...][None, :, :], axis=1)
    p = jnp.exp(jnp.log(yg) * _Q)
    part = jnp.sum(((1.0 - p) * (1.0 / _Q) - _CONST) * w) * (1.0 / _B)

    @pl.when(i == 0)
    def _():
        out_ref[0, 0] = 0.0

    out_ref[0, 0] += part


def kernel(logits, targets, indexes, weight):
    idx = indexes.astype(jnp.int32)
    tgt = targets.astype(jnp.int32)
    lt = logits.T
    blk, w = _sc_gather(lt, weight.T, tgt, idx)
    out = pl.pallas_call(
        _loss_body,
        grid=(_GRID,),
        in_specs=[
            pl.BlockSpec((_CHUNK, _CHUNK), lambda i: (0, 0)),
            pl.BlockSpec((_NBLK // _GRID, _CHUNK, _CHUNK), lambda i: (i, 0, 0)),
            pl.BlockSpec((_NBLK // _GRID, _CHUNK), lambda i: (i, 0)),
        ],
        out_specs=pl.BlockSpec((1, 1), lambda i: (0, 0),
                               memory_space=pltpu.SMEM),
        out_shape=jax.ShapeDtypeStruct((1, 1), jnp.float32),
    )(jnp.eye(_CHUNK, dtype=jnp.float32), blk, w.reshape(_NBLK, _CHUNK))
    return out[0, 0]


# speedup vs baseline: 2.0866x; 1.1798x over previous
import functools

import jax
import jax.numpy as jnp
from jax import lax
from jax.experimental import pallas as pl
from jax.experimental.pallas import tpu as pltpu
from jax.experimental.pallas import tpu_sc as plsc

_Q = 0.7
_K = 0.5
_B = 16384
_NCLS = 1000
_NCORES = 2
_NSUB = 16
_NW = _NCORES * _NSUB
_PER_W = _B // _NW
_CHUNK = 128
_NCH = _PER_W // _CHUNK
_LANES = 16
_TRAIN = 1000000
_CONST = (1.0 - _K ** _Q) / _Q


def _sc_gather(lt, weight_row, targets, indexes):
    mesh = plsc.VectorSubcoreMesh(core_axis_name="c", subcore_axis_name="s")

    @functools.partial(
        pl.kernel,
        mesh=mesh,
        out_type=(
            jax.ShapeDtypeStruct((_B,), jnp.float32),
            jax.ShapeDtypeStruct((_B,), jnp.float32),
        ),
        scratch_types=[
            pltpu.VMEM((_PER_W,), jnp.int32),
            pltpu.VMEM((_PER_W,), jnp.int32),
            pltpu.VMEM((_CHUNK, _CHUNK), jnp.float32),
            pltpu.VMEM((_CHUNK, _CHUNK), jnp.float32),
            pltpu.VMEM((_CHUNK, _CHUNK), jnp.float32),
            pltpu.VMEM((_CHUNK, _CHUNK), jnp.float32),
            pltpu.VMEM((_PER_W,), jnp.float32),
            pltpu.VMEM((_PER_W,), jnp.float32),
            pltpu.VMEM_SHARED((_TRAIN,), jnp.float32),
            pltpu.SemaphoreType.DMA,
            pltpu.SemaphoreType.DMA,
        ],
    )
    def gather_kernel(lt_hbm, w_hbm, t_hbm, i_hbm, yg_out, w_out,
                      tgt_v, widx_v, b0, b1, b2, b3, yg_v, w_v, w_sh,
                      sem_g, sem_w):
        wid = lax.axis_index("c") * _NSUB + lax.axis_index("s")
        base = wid * _PER_W
        pltpu.sync_copy(t_hbm.at[pl.ds(base, _PER_W)], tgt_v)
        pltpu.sync_copy(i_hbm.at[pl.ds(base, _PER_W)], widx_v)
        bufs = (b0, b1, b2, b3)
        gcopies = []
        for g in range(_NCH):
            gcopies.append(pltpu.async_copy(
                lt_hbm.at[tgt_v.at[pl.ds(g * _CHUNK, _CHUNK)],
                          pl.ds(base + g * _CHUNK, _CHUNK)],
                bufs[g], sem_g))

        @pl.when(lax.axis_index("s") == 0)
        def _():
            pltpu.sync_copy(w_hbm.at[0], w_sh)

        plsc.subcore_barrier()
        wcopies = []
        for c in range(_NCH):
            sl = pl.ds(c * _CHUNK, _CHUNK)
            wcopies.append(pltpu.async_copy(
                w_sh.at[widx_v.at[sl]], w_v.at[sl], sem_w))
        lane = lax.iota(jnp.int32, _LANES)
        for g in range(_NCH):
            gcopies[g].wait()
            buf = bufs[g]
            for j in range(_CHUNK // _LANES):
                acc = jnp.zeros((_LANES,), jnp.float32)
                for r in range(_LANES):
                    row = buf[_LANES * j + r, pl.ds(_LANES * j, _LANES)]
                    acc = acc + jnp.where(lane == r, row, 0.0)
                yg_v[pl.ds(g * _CHUNK + _LANES * j, _LANES)] = acc
        for cp in wcopies:
            cp.wait()
        pltpu.sync_copy(yg_v, yg_out.at[pl.ds(base, _PER_W)])
        pltpu.sync_copy(w_v, w_out.at[pl.ds(base, _PER_W)])

    return gather_kernel(lt, weight_row, targets, indexes)


def _loss_body(yg_ref, w_ref, out_ref):
    yg = yg_ref[...]
    w = w_ref[...]
    p = jnp.exp(jnp.log(yg) * _Q)
    out_ref[0, 0] = jnp.sum(((1.0 - p) * (1.0 / _Q) - _CONST) * w) * (1.0 / _B)


def kernel(logits, targets, indexes, weight):
    idx = indexes.astype(jnp.int32)
    tgt = targets.astype(jnp.int32)
    lt = logits.T
    yg, w = _sc_gather(lt, weight.T, tgt, idx)
    out = pl.pallas_call(
        _loss_body,
        out_shape=jax.ShapeDtypeStruct((1, 1), jnp.float32),
        out_specs=pl.BlockSpec(memory_space=pltpu.SMEM),
    )(yg, w)
    return out[0, 0]
